# baseline (speedup 1.0000x reference)
import functools

import jax
import jax.numpy as jnp
from jax import lax
from jax.experimental import pallas as pl
from jax.experimental.pallas import tpu as pltpu
from jax.experimental.pallas import tpu_sc as plsc

NC = 2
NS = 16
NW = NC * NS
L = 16
D = 32
IDX_CHUNK = 128


def _sc_kernel(b_per_w, n_chunks, uid_hbm, mid_hbm, uemb_hbm, memb_hbm,
               ubias_hbm, mbias_hbm, gbias_hbm, out_hbm,
               uid_v, mid_v, urows_v, mrows_v, ub_v, mb_v, gb_v, out_v, sem):
    wid = lax.axis_index("s") * NC + lax.axis_index("c")
    base = wid * b_per_w

    pltpu.sync_copy(uid_hbm.at[wid], uid_v)
    pltpu.sync_copy(mid_hbm.at[wid], mid_v)
    pltpu.sync_copy(gbias_hbm, gb_v)

    copies = []
    for j in range(n_chunks):
        rows = pl.ds(j * IDX_CHUNK, IDX_CHUNK)
        copies.append(pltpu.async_copy(uemb_hbm.at[uid_v.at[j]], urows_v.at[rows], sem))
        copies.append(pltpu.async_copy(memb_hbm.at[mid_v.at[j]], mrows_v.at[rows], sem))
        copies.append(pltpu.async_copy(ubias_hbm.at[uid_v.at[j]], ub_v.at[rows], sem))
        copies.append(pltpu.async_copy(mbias_hbm.at[mid_v.at[j]], mb_v.at[rows], sem))
    for c in copies:
        c.wait()

    gb = gb_v[...]

    @pl.loop(0, b_per_w, step=L)
    def _(i):
        rows16 = i + lax.iota(jnp.int32, L)
        acc = ub_v[pl.ds(i, L)] + mb_v[pl.ds(i, L)] + gb
        for d in range(D):
            dsplat = jnp.full((L,), d, jnp.int32)
            acc = acc + (plsc.load_gather(urows_v, [rows16, dsplat])
                         * plsc.load_gather(mrows_v, [rows16, dsplat]))
        out_v[pl.ds(i, L)] = acc

    pltpu.sync_copy(out_v, out_hbm.at[pl.ds(base, b_per_w)])


def kernel(user_ids, movie_ids, user_emb_table, movie_emb_table,
           user_bias_table, movie_bias_table, global_bias):
    B = user_ids.shape[0]
    assert B % (NW * IDX_CHUNK) == 0
    b_per_w = B // NW
    n_chunks = b_per_w // IDX_CHUNK

    uid = user_ids.astype(jnp.int32).reshape(NW, n_chunks, IDX_CHUNK)
    mid = movie_ids.astype(jnp.int32).reshape(NW, n_chunks, IDX_CHUNK)
    gb16 = jnp.broadcast_to(global_bias.astype(jnp.float32), (L,))
    ubias_flat = user_bias_table.reshape(-1)
    mbias_flat = movie_bias_table.reshape(-1)

    mesh = plsc.VectorSubcoreMesh(core_axis_name="c", subcore_axis_name="s")
    body = functools.partial(_sc_kernel, b_per_w, n_chunks)
    run = pl.kernel(
        body,
        out_type=jax.ShapeDtypeStruct((B,), jnp.float32),
        mesh=mesh,
        compiler_params=pltpu.CompilerParams(
            needs_layout_passes=False, use_tc_tiling_on_sc=False),
        scratch_types=[
            pltpu.VMEM((n_chunks, IDX_CHUNK), jnp.int32),
            pltpu.VMEM((n_chunks, IDX_CHUNK), jnp.int32),
            pltpu.VMEM((b_per_w, D), jnp.float32),
            pltpu.VMEM((b_per_w, D), jnp.float32),
            pltpu.VMEM((b_per_w,), jnp.float32),
            pltpu.VMEM((b_per_w,), jnp.float32),
            pltpu.VMEM((L,), jnp.float32),
            pltpu.VMEM((b_per_w,), jnp.float32),
            pltpu.SemaphoreType.DMA,
        ],
    )
    return run(uid, mid, user_emb_table, movie_emb_table,
               ubias_flat, mbias_flat, gb16)
